# Initial kernel scaffold; baseline (speedup 1.0000x reference)
import jax
import jax.numpy as jnp
from jax.experimental import pallas as pl


def kernel(x, edge_index, edge_weight, W1, b1, W2, b2):
    raise NotImplementedError("write your pallas kernel here")


# speedup vs baseline: 19.8811x; 19.8811x over previous
import functools

import jax
import jax.numpy as jnp
from jax import lax
from jax.experimental import pallas as pl
from jax.experimental.pallas import tpu as pltpu
from jax.experimental.pallas import tpu_sc as plsc

N = 10000
F_IN = 128
HID = 16
C = 16
L = 16
NC = 2
NS = 16
NW = NC * NS
CHUNK = 128
N_PAD = 10240
ZR = N_PAD // NS

_mesh = plsc.VectorSubcoreMesh(core_axis_name="c", subcore_axis_name="s")
_sc_params = pltpu.CompilerParams(needs_layout_passes=False, use_tc_tiling_on_sc=False)


def _iota16():
    return lax.iota(jnp.int32, 16)


def _full16(v):
    return jnp.full((16,), v, jnp.int32)


def _make_deg(cpt):
    @functools.partial(
        pl.kernel,
        mesh=_mesh,
        compiler_params=_sc_params,
        out_type=jax.ShapeDtypeStruct((NC, N_PAD), jnp.float32),
        scratch_types=[
            pltpu.VMEM((cpt, CHUNK), jnp.int32),
            pltpu.VMEM((cpt, CHUNK), jnp.float32),
            pltpu.VMEM((ZR,), jnp.float32),
            pltpu.VMEM_SHARED((N_PAD,), jnp.float32),
        ],
    )
    def deg_kernel(row_hbm, ew_hbm, zeros_hbm, out_hbm, idx_v, val_v, zs_v, acc_sh):
        cid = lax.axis_index("c")
        sid = lax.axis_index("s")
        wid = sid * NC + cid
        pltpu.sync_copy(zeros_hbm.at[pl.ds(0, ZR)], zs_v)
        pltpu.sync_copy(zs_v, acc_sh.at[pl.ds(sid * ZR, ZR)])
        plsc.subcore_barrier()
        pltpu.sync_copy(row_hbm.at[wid], idx_v)
        pltpu.sync_copy(ew_hbm.at[wid], val_v)

        def body(j, carry):
            pltpu.sync_copy(val_v.at[j], acc_sh.at[idx_v.at[j]], add=True)
            return carry

        lax.fori_loop(0, cpt, body, 0)
        plsc.subcore_barrier()
        pltpu.sync_copy(acc_sh.at[pl.ds(sid * ZR, ZR)], zs_v)
        pltpu.sync_copy(zs_v, out_hbm.at[cid, pl.ds(sid * ZR, ZR)])

    return deg_kernel


def _make_agg(cpt, compute_wn):
    out_type = [jax.ShapeDtypeStruct((NC, N_PAD, HID), jnp.float32)]
    if compute_wn:
        out_type.append(jax.ShapeDtypeStruct((NW, cpt, CHUNK), jnp.float32))

    scratch = [
        pltpu.VMEM((cpt, CHUNK), jnp.int32),
        pltpu.VMEM((cpt, CHUNK), jnp.int32),
        pltpu.VMEM((cpt, CHUNK), jnp.float32),
        pltpu.VMEM((CHUNK, HID), jnp.float32),
        pltpu.VMEM((CHUNK, HID), jnp.float32),
        pltpu.VMEM((ZR, HID), jnp.float32),
        pltpu.VMEM_SHARED((N_PAD, HID), jnp.float32),
        pltpu.SemaphoreType.DMA,
    ]
    if compute_wn:
        scratch.append(pltpu.VMEM((cpt, CHUNK), jnp.float32))
        scratch.append(pltpu.VMEM((N_PAD,), jnp.float32))

    @functools.partial(
        pl.kernel,
        mesh=_mesh,
        compiler_params=_sc_params,
        out_type=tuple(out_type) if compute_wn else out_type[0],
        scratch_types=scratch,
    )
    def agg_kernel(*refs):
        if compute_wn:
            (row_hbm, col_hbm, ew_hbm, dis_hbm, y_hbm, zeros_hbm,
             acc_out, wn_out,
             row_v, col_v, wn_v, rows_v, sc_v, st_v, acc_sh, sem,
             ew_v, dis_v) = refs
        else:
            (row_hbm, col_hbm, wn_hbm, y_hbm, zeros_hbm,
             acc_out,
             row_v, col_v, wn_v, rows_v, sc_v, st_v, acc_sh, sem) = refs
        cid = lax.axis_index("c")
        sid = lax.axis_index("s")
        wid = sid * NC + cid

        pltpu.sync_copy(zeros_hbm, st_v)
        pltpu.sync_copy(st_v, acc_sh.at[pl.ds(sid * ZR, ZR)])
        plsc.subcore_barrier()

        pltpu.sync_copy(row_hbm.at[wid], row_v)
        pltpu.sync_copy(col_hbm.at[wid], col_v)
        if compute_wn:
            pltpu.sync_copy(ew_hbm.at[wid], ew_v)
            pltpu.sync_copy(dis_hbm, dis_v)
        else:
            pltpu.sync_copy(wn_hbm.at[wid], wn_v)

        def agg_body(j, carry):
            if compute_wn:
                for g in range(CHUNK // L):
                    sl = pl.ds(g * L, L)
                    dr = plsc.load_gather(dis_v, [row_v[j, sl]])
                    dc = plsc.load_gather(dis_v, [col_v[j, sl]])
                    wn_v[j, sl] = dr * ew_v[j, sl] * dc
            pltpu.async_copy(y_hbm.at[col_v.at[j]], rows_v, sem).wait()
            for i in range(CHUNK):
                wb = plsc.load_gather(wn_v, [_full16(j), _full16(i)])
                sc_v[i, :] = rows_v[i, :] * wb
            pltpu.sync_copy(sc_v, acc_sh.at[row_v.at[j]], add=True)
            return carry

        lax.fori_loop(0, cpt, agg_body, 0)
        if compute_wn:
            pltpu.sync_copy(wn_v, wn_out.at[wid])
        plsc.subcore_barrier()
        pltpu.sync_copy(acc_sh.at[pl.ds(sid * ZR, ZR)], st_v)
        pltpu.sync_copy(st_v, acc_out.at[cid, pl.ds(sid * ZR, ZR)])

    return agg_kernel


def _mm_kernel(x_ref, w_ref, o_ref):
    o_ref[...] = jnp.dot(x_ref[...], w_ref[...],
                         preferred_element_type=jnp.float32)


def _dis_kernel(deg_ref, o_ref):
    d = deg_ref[0, :] + deg_ref[1, :]
    o_ref[0, :] = jnp.where(
        d > 0.0, lax.rsqrt(jnp.maximum(d, 1e-30)), 0.0)


def _layer_out_kernel(t0_ref, acc_ref, b_ref, w_ref, o_ref):
    h = t0_ref[...] - (acc_ref[0] + acc_ref[1]) + b_ref[...]
    h = jnp.maximum(h, 0.0)
    o_ref[...] = jnp.dot(h, w_ref[...], preferred_element_type=jnp.float32)


def _final_kernel(u0_ref, acc_ref, b_ref, o_ref):
    z = u0_ref[...] - (acc_ref[0] + acc_ref[1]) + b_ref[...]
    z = z - jnp.max(z, axis=1, keepdims=True)
    o_ref[...] = z - jnp.log(jnp.sum(jnp.exp(z), axis=1, keepdims=True))


_ROWS_BLK = 2000


def _tc_matmul(x, w):
    m, k = x.shape
    n = w.shape[1]
    grid = m // _ROWS_BLK
    return pl.pallas_call(
        _mm_kernel,
        grid=(grid,),
        in_specs=[
            pl.BlockSpec((_ROWS_BLK, k), lambda i: (i, 0)),
            pl.BlockSpec((k, n), lambda i: (0, 0)),
        ],
        out_specs=pl.BlockSpec((_ROWS_BLK, n), lambda i: (i, 0)),
        out_shape=jax.ShapeDtypeStruct((m, n), jnp.float32),
    )(x, w)


def _tc_dis(deg_p):
    return pl.pallas_call(
        _dis_kernel,
        out_shape=jax.ShapeDtypeStruct((1, N_PAD), jnp.float32),
    )(deg_p)


def _tc_layer_out(t0, acc_p, b, w):
    m = t0.shape[0]
    n = w.shape[1]
    grid = m // _ROWS_BLK
    return pl.pallas_call(
        _layer_out_kernel,
        grid=(grid,),
        in_specs=[
            pl.BlockSpec((_ROWS_BLK, HID), lambda i: (i, 0)),
            pl.BlockSpec((NC, _ROWS_BLK, HID), lambda i: (0, i, 0)),
            pl.BlockSpec((1, HID), lambda i: (0, 0)),
            pl.BlockSpec((HID, n), lambda i: (0, 0)),
        ],
        out_specs=pl.BlockSpec((_ROWS_BLK, n), lambda i: (i, 0)),
        out_shape=jax.ShapeDtypeStruct((m, n), jnp.float32),
    )(t0, acc_p, b, w)


def _tc_final(u0, acc_p, b):
    m = u0.shape[0]
    grid = m // _ROWS_BLK
    return pl.pallas_call(
        _final_kernel,
        grid=(grid,),
        in_specs=[
            pl.BlockSpec((_ROWS_BLK, C), lambda i: (i, 0)),
            pl.BlockSpec((NC, _ROWS_BLK, C), lambda i: (0, i, 0)),
            pl.BlockSpec((1, C), lambda i: (0, 0)),
        ],
        out_specs=pl.BlockSpec((_ROWS_BLK, C), lambda i: (i, 0)),
        out_shape=jax.ShapeDtypeStruct((m, C), jnp.float32),
    )(u0, acc_p, b)


def kernel(x, edge_index, edge_weight, W1, b1, W2, b2):
    e = edge_index.shape[1]
    cpt = -(-e // (NW * CHUNK))
    e_pad = NW * cpt * CHUNK
    npad = e_pad - e

    row = edge_index[0]
    col = edge_index[1]
    k = jnp.arange(npad, dtype=jnp.int32)
    row_p = jnp.concatenate([row, N + (k % (N_PAD - N))]).reshape(NW, cpt, CHUNK)
    col_p = jnp.concatenate([col, (k * 97) % N]).reshape(NW, cpt, CHUNK)
    ew_p = jnp.concatenate(
        [edge_weight, jnp.zeros((npad,), jnp.float32)]).reshape(NW, cpt, CHUNK)
    zeros = jnp.zeros((ZR, HID), jnp.float32)

    deg_p = _make_deg(cpt)(row_p, ew_p, zeros[:, 0])
    dis = _tc_dis(deg_p)[0]

    w1c = jnp.concatenate([W1[0], W1[1]], axis=1)
    t0y1 = _tc_matmul(x, w1c)
    t0 = t0y1[:, :HID]
    y1 = t0y1[:, HID:]

    acc1_p, wn = _make_agg(cpt, True)(row_p, col_p, ew_p, dis, y1, zeros)

    w2c = jnp.concatenate([W2[0], W2[1]], axis=1)
    u0y2 = _tc_layer_out(t0, acc1_p[:, :N, :], b1.reshape(1, HID), w2c)
    u0 = u0y2[:, :C]
    y2 = u0y2[:, C:]

    acc2_p = _make_agg(cpt, False)(row_p, col_p, wn, y2, zeros)

    return _tc_final(u0, acc2_p[:, :N, :], b2.reshape(1, C))


# speedup vs baseline: 33.4399x; 1.6820x over previous
import functools

import jax
import jax.numpy as jnp
from jax import lax
from jax.experimental import pallas as pl
from jax.experimental.pallas import tpu as pltpu
from jax.experimental.pallas import tpu_sc as plsc

N = 10000
F_IN = 128
HID = 16
C = 16
L = 16
NC = 2
NS = 16
NW = NC * NS
CHUNK = 128
N_PAD = 10240
ZR = N_PAD // NS

_mesh = plsc.VectorSubcoreMesh(core_axis_name="c", subcore_axis_name="s")
_sc_params = pltpu.CompilerParams(needs_layout_passes=False, use_tc_tiling_on_sc=False)


def _iota16():
    return lax.iota(jnp.int32, 16)


def _full16(v):
    return jnp.full((16,), v, jnp.int32)


def _make_deg(cpt):
    @functools.partial(
        pl.kernel,
        mesh=_mesh,
        compiler_params=_sc_params,
        out_type=jax.ShapeDtypeStruct((NC, N_PAD), jnp.float32),
        scratch_types=[
            pltpu.VMEM((cpt, CHUNK), jnp.int32),
            pltpu.VMEM((cpt, CHUNK), jnp.float32),
            pltpu.VMEM((ZR,), jnp.float32),
            pltpu.VMEM_SHARED((N_PAD,), jnp.float32),
        ],
    )
    def deg_kernel(row_hbm, ew_hbm, zeros_hbm, out_hbm, idx_v, val_v, zs_v, acc_sh):
        cid = lax.axis_index("c")
        sid = lax.axis_index("s")
        wid = sid * NC + cid
        pltpu.sync_copy(zeros_hbm.at[pl.ds(0, ZR)], zs_v)
        pltpu.sync_copy(zs_v, acc_sh.at[pl.ds(sid * ZR, ZR)])
        plsc.subcore_barrier()
        pltpu.sync_copy(row_hbm.at[wid], idx_v)
        pltpu.sync_copy(ew_hbm.at[wid], val_v)

        def body(j, carry):
            pltpu.sync_copy(val_v.at[j], acc_sh.at[idx_v.at[j]], add=True)
            return carry

        lax.fori_loop(0, cpt, body, 0)
        plsc.subcore_barrier()
        pltpu.sync_copy(acc_sh.at[pl.ds(sid * ZR, ZR)], zs_v)
        pltpu.sync_copy(zs_v, out_hbm.at[cid, pl.ds(sid * ZR, ZR)])

    return deg_kernel


def _make_agg(cpt, compute_wn):
    out_type = [jax.ShapeDtypeStruct((NC, N_PAD, HID), jnp.float32)]
    if compute_wn:
        out_type.append(jax.ShapeDtypeStruct((NW, cpt, CHUNK), jnp.float32))

    scratch = [
        pltpu.VMEM((cpt, CHUNK), jnp.int32),
        pltpu.VMEM((cpt, CHUNK), jnp.int32),
        pltpu.VMEM((cpt, CHUNK), jnp.float32),
        pltpu.VMEM((CHUNK, HID), jnp.float32),
        pltpu.VMEM((CHUNK, HID), jnp.float32),
        pltpu.VMEM((CHUNK, HID), jnp.float32),
        pltpu.VMEM((CHUNK, HID), jnp.float32),
        pltpu.VMEM((ZR, HID), jnp.float32),
        pltpu.VMEM_SHARED((N_PAD, HID), jnp.float32),
        pltpu.SemaphoreType.DMA,
        pltpu.SemaphoreType.DMA,
    ]
    if compute_wn:
        scratch.append(pltpu.VMEM((cpt, CHUNK), jnp.float32))
        scratch.append(pltpu.VMEM((N_PAD,), jnp.float32))

    @functools.partial(
        pl.kernel,
        mesh=_mesh,
        compiler_params=_sc_params,
        out_type=tuple(out_type) if compute_wn else out_type[0],
        scratch_types=scratch,
    )
    def agg_kernel(*refs):
        if compute_wn:
            (row_hbm, col_hbm, ew_hbm, dis_hbm, y_hbm, zeros_hbm,
             acc_out, wn_out,
             row_v, col_v, wn_v, rows_a, rows_b, sc_a, sc_b, st_v, acc_sh,
             sem_a, sem_b, ew_v, dis_v) = refs
        else:
            (row_hbm, col_hbm, wn_hbm, y_hbm, zeros_hbm,
             acc_out,
             row_v, col_v, wn_v, rows_a, rows_b, sc_a, sc_b, st_v, acc_sh,
             sem_a, sem_b) = refs
        cid = lax.axis_index("c")
        sid = lax.axis_index("s")
        wid = sid * NC + cid

        pltpu.sync_copy(zeros_hbm, st_v)
        pltpu.sync_copy(st_v, acc_sh.at[pl.ds(sid * ZR, ZR)])
        plsc.subcore_barrier()

        pltpu.sync_copy(row_hbm.at[wid], row_v)
        pltpu.sync_copy(col_hbm.at[wid], col_v)
        if compute_wn:
            pltpu.sync_copy(ew_hbm.at[wid], ew_v)
            pltpu.sync_copy(dis_hbm, dis_v)
        else:
            pltpu.sync_copy(wn_hbm.at[wid], wn_v)

        def compute_wn_chunk(j):
            @plsc.parallel_loop(0, CHUNK // L)
            def _(g):
                sl = pl.ds(g * L, L)
                dr = plsc.load_gather(dis_v, [row_v[j, sl]])
                dc = plsc.load_gather(dis_v, [col_v[j, sl]])
                wn_v[j, sl] = dr * ew_v[j, sl] * dc

        def scale_scatter(j, rows_v, sc_v):
            @plsc.parallel_loop(0, CHUNK, unroll=4)
            def _(i):
                wb = plsc.load_gather(wn_v, [_full16(j), _full16(i)])
                sc_v[i, :] = rows_v[i, :] * wb

            pltpu.sync_copy(sc_v, acc_sh.at[row_v.at[j]], add=True)

        pltpu.async_copy(y_hbm.at[col_v.at[0]], rows_a, sem_a)

        def agg_body(p, carry):
            j0 = 2 * p
            j1 = j0 + 1
            pltpu.async_copy(y_hbm.at[col_v.at[j1]], rows_b, sem_b)
            if compute_wn:
                compute_wn_chunk(j0)
            pltpu.make_async_copy(y_hbm.at[col_v.at[j0]], rows_a, sem_a).wait()
            scale_scatter(j0, rows_a, sc_a)

            @pl.when(j1 + 1 < cpt)
            def _():
                pltpu.async_copy(y_hbm.at[col_v.at[j1 + 1]], rows_a, sem_a)

            if compute_wn:
                compute_wn_chunk(j1)
            pltpu.make_async_copy(y_hbm.at[col_v.at[j1]], rows_b, sem_b).wait()
            scale_scatter(j1, rows_b, sc_b)
            return carry

        lax.fori_loop(0, cpt // 2, agg_body, 0)
        if compute_wn:
            pltpu.sync_copy(wn_v, wn_out.at[wid])
        plsc.subcore_barrier()
        pltpu.sync_copy(acc_sh.at[pl.ds(sid * ZR, ZR)], st_v)
        pltpu.sync_copy(st_v, acc_out.at[cid, pl.ds(sid * ZR, ZR)])

    return agg_kernel


def _mm_kernel(x_ref, w_ref, o_ref):
    o_ref[...] = jnp.dot(x_ref[...], w_ref[...],
                         preferred_element_type=jnp.float32)


def _dis_kernel(deg_ref, o_ref):
    d = deg_ref[0, :] + deg_ref[1, :]
    o_ref[0, :] = jnp.where(
        d > 0.0, lax.rsqrt(jnp.maximum(d, 1e-30)), 0.0)


def _layer_out_kernel(t0_ref, acc_ref, b_ref, w_ref, o_ref):
    h = t0_ref[...] - (acc_ref[0] + acc_ref[1]) + b_ref[...]
    h = jnp.maximum(h, 0.0)
    o_ref[...] = jnp.dot(h, w_ref[...], preferred_element_type=jnp.float32)


def _final_kernel(u0_ref, acc_ref, b_ref, o_ref):
    z = u0_ref[...] - (acc_ref[0] + acc_ref[1]) + b_ref[...]
    z = z - jnp.max(z, axis=1, keepdims=True)
    o_ref[...] = z - jnp.log(jnp.sum(jnp.exp(z), axis=1, keepdims=True))


_ROWS_BLK = 2000


def _tc_matmul(x, w):
    m, k = x.shape
    n = w.shape[1]
    grid = m // _ROWS_BLK
    return pl.pallas_call(
        _mm_kernel,
        grid=(grid,),
        in_specs=[
            pl.BlockSpec((_ROWS_BLK, k), lambda i: (i, 0)),
            pl.BlockSpec((k, n), lambda i: (0, 0)),
        ],
        out_specs=pl.BlockSpec((_ROWS_BLK, n), lambda i: (i, 0)),
        out_shape=jax.ShapeDtypeStruct((m, n), jnp.float32),
    )(x, w)


def _tc_dis(deg_p):
    return pl.pallas_call(
        _dis_kernel,
        out_shape=jax.ShapeDtypeStruct((1, N_PAD), jnp.float32),
    )(deg_p)


def _tc_layer_out(t0, acc_p, b, w):
    m = t0.shape[0]
    n = w.shape[1]
    grid = m // _ROWS_BLK
    return pl.pallas_call(
        _layer_out_kernel,
        grid=(grid,),
        in_specs=[
            pl.BlockSpec((_ROWS_BLK, HID), lambda i: (i, 0)),
            pl.BlockSpec((NC, _ROWS_BLK, HID), lambda i: (0, i, 0)),
            pl.BlockSpec((1, HID), lambda i: (0, 0)),
            pl.BlockSpec((HID, n), lambda i: (0, 0)),
        ],
        out_specs=pl.BlockSpec((_ROWS_BLK, n), lambda i: (i, 0)),
        out_shape=jax.ShapeDtypeStruct((m, n), jnp.float32),
    )(t0, acc_p, b, w)


def _tc_final(u0, acc_p, b):
    m = u0.shape[0]
    grid = m // _ROWS_BLK
    return pl.pallas_call(
        _final_kernel,
        grid=(grid,),
        in_specs=[
            pl.BlockSpec((_ROWS_BLK, C), lambda i: (i, 0)),
            pl.BlockSpec((NC, _ROWS_BLK, C), lambda i: (0, i, 0)),
            pl.BlockSpec((1, C), lambda i: (0, 0)),
        ],
        out_specs=pl.BlockSpec((_ROWS_BLK, C), lambda i: (i, 0)),
        out_shape=jax.ShapeDtypeStruct((m, C), jnp.float32),
    )(u0, acc_p, b)


def kernel(x, edge_index, edge_weight, W1, b1, W2, b2):
    e = edge_index.shape[1]
    cpt = -(-e // (NW * CHUNK))
    cpt += cpt % 2
    e_pad = NW * cpt * CHUNK
    npad = e_pad - e

    row = edge_index[0]
    col = edge_index[1]
    k = jnp.arange(npad, dtype=jnp.int32)
    row_p = jnp.concatenate([row, N + (k % (N_PAD - N))]).reshape(NW, cpt, CHUNK)
    col_p = jnp.concatenate([col, (k * 97) % N]).reshape(NW, cpt, CHUNK)
    ew_p = jnp.concatenate(
        [edge_weight, jnp.zeros((npad,), jnp.float32)]).reshape(NW, cpt, CHUNK)
    zeros = jnp.zeros((ZR, HID), jnp.float32)

    deg_p = _make_deg(cpt)(row_p, ew_p, zeros[:, 0])
    dis = _tc_dis(deg_p)[0]

    w1c = jnp.concatenate([W1[0], W1[1]], axis=1)
    t0y1 = _tc_matmul(x, w1c)
    t0 = t0y1[:, :HID]
    y1 = t0y1[:, HID:]

    acc1_p, wn = _make_agg(cpt, True)(row_p, col_p, ew_p, dis, y1, zeros)

    w2c = jnp.concatenate([W2[0], W2[1]], axis=1)
    u0y2 = _tc_layer_out(t0, acc1_p[:, :N, :], b1.reshape(1, HID), w2c)
    u0 = u0y2[:, :C]
    y2 = u0y2[:, C:]

    acc2_p = _make_agg(cpt, False)(row_p, col_p, wn, y2, zeros)

    return _tc_final(u0, acc2_p[:, :N, :], b2.reshape(1, C))


# speedup vs baseline: 34.8014x; 1.0407x over previous
import functools

import jax
import jax.numpy as jnp
from jax import lax
from jax.experimental import pallas as pl
from jax.experimental.pallas import tpu as pltpu
from jax.experimental.pallas import tpu_sc as plsc

N = 10000
F_IN = 128
HID = 16
C = 16
L = 16
NC = 2
NS = 16
NW = NC * NS
CHUNK = 128
N_PAD = 10240
ZR = N_PAD // NS

_mesh = plsc.VectorSubcoreMesh(core_axis_name="c", subcore_axis_name="s")
_sc_params = pltpu.CompilerParams(needs_layout_passes=False, use_tc_tiling_on_sc=False)


def _iota16():
    return lax.iota(jnp.int32, 16)


def _full16(v):
    return jnp.full((16,), v, jnp.int32)


def _make_deg(cpt):
    @functools.partial(
        pl.kernel,
        mesh=_mesh,
        compiler_params=_sc_params,
        out_type=jax.ShapeDtypeStruct((NC, N_PAD), jnp.float32),
        scratch_types=[
            pltpu.VMEM((cpt, CHUNK), jnp.int32),
            pltpu.VMEM((cpt, CHUNK), jnp.float32),
            pltpu.VMEM((ZR,), jnp.float32),
            pltpu.VMEM_SHARED((N_PAD,), jnp.float32),
        ],
    )
    def deg_kernel(row_hbm, ew_hbm, zeros_hbm, out_hbm, idx_v, val_v, zs_v, acc_sh):
        cid = lax.axis_index("c")
        sid = lax.axis_index("s")
        wid = sid * NC + cid
        pltpu.sync_copy(zeros_hbm.at[pl.ds(0, ZR)], zs_v)
        pltpu.sync_copy(zs_v, acc_sh.at[pl.ds(sid * ZR, ZR)])
        plsc.subcore_barrier()
        pltpu.sync_copy(row_hbm.at[wid], idx_v)
        pltpu.sync_copy(ew_hbm.at[wid], val_v)

        def body(j, carry):
            pltpu.sync_copy(val_v.at[j], acc_sh.at[idx_v.at[j]], add=True)
            return carry

        lax.fori_loop(0, cpt, body, 0)
        plsc.subcore_barrier()
        pltpu.sync_copy(acc_sh.at[pl.ds(sid * ZR, ZR)], zs_v)
        pltpu.sync_copy(zs_v, out_hbm.at[cid, pl.ds(sid * ZR, ZR)])

    return deg_kernel


def _make_agg(cpt, compute_wn):
    out_type = [jax.ShapeDtypeStruct((NC, N_PAD, HID), jnp.float32)]
    if compute_wn:
        out_type.append(jax.ShapeDtypeStruct((NW, cpt, CHUNK), jnp.float32))

    scratch = [
        pltpu.VMEM((cpt, CHUNK), jnp.int32),
        pltpu.VMEM((cpt, CHUNK), jnp.int32),
        pltpu.VMEM((cpt, CHUNK), jnp.float32),
        pltpu.VMEM((CHUNK, HID), jnp.float32),
        pltpu.VMEM((CHUNK, HID), jnp.float32),
        pltpu.VMEM((CHUNK, HID), jnp.float32),
        pltpu.VMEM((CHUNK, HID), jnp.float32),
        pltpu.VMEM((ZR, HID), jnp.float32),
        pltpu.VMEM_SHARED((N_PAD, HID), jnp.float32),
        pltpu.SemaphoreType.DMA,
        pltpu.SemaphoreType.DMA,
        pltpu.SemaphoreType.DMA,
        pltpu.SemaphoreType.DMA,
    ]
    if compute_wn:
        scratch.append(pltpu.VMEM((cpt, CHUNK), jnp.float32))
        scratch.append(pltpu.VMEM((N_PAD,), jnp.float32))

    @functools.partial(
        pl.kernel,
        mesh=_mesh,
        compiler_params=_sc_params,
        out_type=tuple(out_type) if compute_wn else out_type[0],
        scratch_types=scratch,
    )
    def agg_kernel(*refs):
        if compute_wn:
            (row_hbm, col_hbm, ew_hbm, dis_hbm, y_hbm, zeros_hbm,
             acc_out, wn_out,
             row_v, col_v, wn_v, rows_a, rows_b, sc_a, sc_b, st_v, acc_sh,
             sem_a, sem_b, sem_sa, sem_sb, ew_v, dis_v) = refs
        else:
            (row_hbm, col_hbm, wn_hbm, y_hbm, zeros_hbm,
             acc_out,
             row_v, col_v, wn_v, rows_a, rows_b, sc_a, sc_b, st_v, acc_sh,
             sem_a, sem_b, sem_sa, sem_sb) = refs
        cid = lax.axis_index("c")
        sid = lax.axis_index("s")
        wid = sid * NC + cid

        pltpu.sync_copy(zeros_hbm, st_v)
        pltpu.sync_copy(st_v, acc_sh.at[pl.ds(sid * ZR, ZR)])
        plsc.subcore_barrier()

        pltpu.sync_copy(row_hbm.at[wid], row_v)
        pltpu.sync_copy(col_hbm.at[wid], col_v)
        if compute_wn:
            pltpu.sync_copy(ew_hbm.at[wid], ew_v)
            pltpu.sync_copy(dis_hbm, dis_v)
        else:
            pltpu.sync_copy(wn_hbm.at[wid], wn_v)

        def compute_wn_chunk(j):
            @plsc.parallel_loop(0, CHUNK // L)
            def _(g):
                sl = pl.ds(g * L, L)
                dr = plsc.load_gather(dis_v, [row_v[j, sl]])
                dc = plsc.load_gather(dis_v, [col_v[j, sl]])
                wn_v[j, sl] = dr * ew_v[j, sl] * dc

        def scale(j, rows_v, sc_v):
            @plsc.parallel_loop(0, CHUNK, unroll=8)
            def _(i):
                wb = plsc.load_gather(wn_v, [_full16(j), _full16(i)])
                sc_v[i, :] = rows_v[i, :] * wb

        pltpu.async_copy(y_hbm.at[col_v.at[0]], rows_a, sem_a)

        def agg_body(p, carry):
            j0 = 2 * p
            j1 = j0 + 1
            pltpu.async_copy(y_hbm.at[col_v.at[j1]], rows_b, sem_b)
            if compute_wn:
                compute_wn_chunk(j0)
            pltpu.make_async_copy(y_hbm.at[col_v.at[j0]], rows_a, sem_a).wait()

            @pl.when(p > 0)
            def _():
                pltpu.make_async_copy(
                    sc_a, acc_sh.at[row_v.at[j0]], sem_sa).wait()

            scale(j0, rows_a, sc_a)
            pltpu.async_copy(sc_a, acc_sh.at[row_v.at[j0]], sem_sa, add=True)

            @pl.when(j1 + 1 < cpt)
            def _():
                pltpu.async_copy(y_hbm.at[col_v.at[j1 + 1]], rows_a, sem_a)

            if compute_wn:
                compute_wn_chunk(j1)
            pltpu.make_async_copy(y_hbm.at[col_v.at[j1]], rows_b, sem_b).wait()

            @pl.when(p > 0)
            def _():
                pltpu.make_async_copy(
                    sc_b, acc_sh.at[row_v.at[j1]], sem_sb).wait()

            scale(j1, rows_b, sc_b)
            pltpu.async_copy(sc_b, acc_sh.at[row_v.at[j1]], sem_sb, add=True)
            return carry

        lax.fori_loop(0, cpt // 2, agg_body, 0)
        pltpu.make_async_copy(sc_a, acc_sh.at[row_v.at[0]], sem_sa).wait()
        pltpu.make_async_copy(sc_b, acc_sh.at[row_v.at[0]], sem_sb).wait()
        if compute_wn:
            pltpu.sync_copy(wn_v, wn_out.at[wid])
        plsc.subcore_barrier()
        pltpu.sync_copy(acc_sh.at[pl.ds(sid * ZR, ZR)], st_v)
        pltpu.sync_copy(st_v, acc_out.at[cid, pl.ds(sid * ZR, ZR)])

    return agg_kernel


def _mm_kernel(x_ref, w_ref, o_ref):
    o_ref[...] = jnp.dot(x_ref[...], w_ref[...],
                         preferred_element_type=jnp.float32)


def _dis_kernel(deg_ref, o_ref):
    d = deg_ref[0, :] + deg_ref[1, :]
    o_ref[0, :] = jnp.where(
        d > 0.0, lax.rsqrt(jnp.maximum(d, 1e-30)), 0.0)


def _layer_out_kernel(t0_ref, acc_ref, b_ref, w_ref, o_ref):
    h = t0_ref[...] - (acc_ref[0] + acc_ref[1]) + b_ref[...]
    h = jnp.maximum(h, 0.0)
    o_ref[...] = jnp.dot(h, w_ref[...], preferred_element_type=jnp.float32)


def _final_kernel(u0_ref, acc_ref, b_ref, o_ref):
    z = u0_ref[...] - (acc_ref[0] + acc_ref[1]) + b_ref[...]
    z = z - jnp.max(z, axis=1, keepdims=True)
    o_ref[...] = z - jnp.log(jnp.sum(jnp.exp(z), axis=1, keepdims=True))


_ROWS_BLK = 2000


def _tc_matmul(x, w):
    m, k = x.shape
    n = w.shape[1]
    grid = m // _ROWS_BLK
    return pl.pallas_call(
        _mm_kernel,
        grid=(grid,),
        in_specs=[
            pl.BlockSpec((_ROWS_BLK, k), lambda i: (i, 0)),
            pl.BlockSpec((k, n), lambda i: (0, 0)),
        ],
        out_specs=pl.BlockSpec((_ROWS_BLK, n), lambda i: (i, 0)),
        out_shape=jax.ShapeDtypeStruct((m, n), jnp.float32),
    )(x, w)


def _tc_dis(deg_p):
    return pl.pallas_call(
        _dis_kernel,
        out_shape=jax.ShapeDtypeStruct((1, N_PAD), jnp.float32),
    )(deg_p)


def _tc_layer_out(t0, acc_p, b, w):
    m = t0.shape[0]
    n = w.shape[1]
    grid = m // _ROWS_BLK
    return pl.pallas_call(
        _layer_out_kernel,
        grid=(grid,),
        in_specs=[
            pl.BlockSpec((_ROWS_BLK, HID), lambda i: (i, 0)),
            pl.BlockSpec((NC, _ROWS_BLK, HID), lambda i: (0, i, 0)),
            pl.BlockSpec((1, HID), lambda i: (0, 0)),
            pl.BlockSpec((HID, n), lambda i: (0, 0)),
        ],
        out_specs=pl.BlockSpec((_ROWS_BLK, n), lambda i: (i, 0)),
        out_shape=jax.ShapeDtypeStruct((m, n), jnp.float32),
    )(t0, acc_p, b, w)


def _tc_final(u0, acc_p, b):
    m = u0.shape[0]
    grid = m // _ROWS_BLK
    return pl.pallas_call(
        _final_kernel,
        grid=(grid,),
        in_specs=[
            pl.BlockSpec((_ROWS_BLK, C), lambda i: (i, 0)),
            pl.BlockSpec((NC, _ROWS_BLK, C), lambda i: (0, i, 0)),
            pl.BlockSpec((1, C), lambda i: (0, 0)),
        ],
        out_specs=pl.BlockSpec((_ROWS_BLK, C), lambda i: (i, 0)),
        out_shape=jax.ShapeDtypeStruct((m, C), jnp.float32),
    )(u0, acc_p, b)


def kernel(x, edge_index, edge_weight, W1, b1, W2, b2):
    e = edge_index.shape[1]
    cpt = -(-e // (NW * CHUNK))
    cpt += cpt % 2
    e_pad = NW * cpt * CHUNK
    npad = e_pad - e

    row = edge_index[0]
    col = edge_index[1]
    k = jnp.arange(npad, dtype=jnp.int32)
    row_p = jnp.concatenate([row, N + (k % (N_PAD - N))]).reshape(NW, cpt, CHUNK)
    col_p = jnp.concatenate([col, (k * 97) % N]).reshape(NW, cpt, CHUNK)
    ew_p = jnp.concatenate(
        [edge_weight, jnp.zeros((npad,), jnp.float32)]).reshape(NW, cpt, CHUNK)
    zeros = jnp.zeros((ZR, HID), jnp.float32)

    deg_p = _make_deg(cpt)(row_p, ew_p, zeros[:, 0])
    dis = _tc_dis(deg_p)[0]

    w1c = jnp.concatenate([W1[0], W1[1]], axis=1)
    t0y1 = _tc_matmul(x, w1c)
    t0 = t0y1[:, :HID]
    y1 = t0y1[:, HID:]

    acc1_p, wn = _make_agg(cpt, True)(row_p, col_p, ew_p, dis, y1, zeros)

    w2c = jnp.concatenate([W2[0], W2[1]], axis=1)
    u0y2 = _tc_layer_out(t0, acc1_p[:, :N, :], b1.reshape(1, HID), w2c)
    u0 = u0y2[:, :C]
    y2 = u0y2[:, C:]

    acc2_p = _make_agg(cpt, False)(row_p, col_p, wn, y2, zeros)

    return _tc_final(u0, acc2_p[:, :N, :], b2.reshape(1, C))


# speedup vs baseline: 38.1444x; 1.0961x over previous
import functools

import jax
import jax.numpy as jnp
from jax import lax
from jax.experimental import pallas as pl
from jax.experimental.pallas import tpu as pltpu
from jax.experimental.pallas import tpu_sc as plsc

N = 10000
F_IN = 128
HID = 16
C = 16
L = 16
NC = 2
NS = 16
NW = NC * NS
CHUNK = 128
N_PAD = 10240
ZR = N_PAD // NS

_mesh = plsc.VectorSubcoreMesh(core_axis_name="c", subcore_axis_name="s")
_sc_params = pltpu.CompilerParams(needs_layout_passes=False, use_tc_tiling_on_sc=False)


def _iota16():
    return lax.iota(jnp.int32, 16)


def _full16(v):
    return jnp.full((16,), v, jnp.int32)


def _make_deg(cpt):
    @functools.partial(
        pl.kernel,
        mesh=_mesh,
        compiler_params=_sc_params,
        out_type=jax.ShapeDtypeStruct((NC, N_PAD), jnp.float32),
        scratch_types=[
            pltpu.VMEM((cpt, CHUNK), jnp.int32),
            pltpu.VMEM((cpt, CHUNK), jnp.float32),
            pltpu.VMEM((ZR,), jnp.float32),
            pltpu.VMEM_SHARED((N_PAD,), jnp.float32),
        ],
    )
    def deg_kernel(row_hbm, ew_hbm, zeros_hbm, out_hbm, idx_v, val_v, zs_v, acc_sh):
        cid = lax.axis_index("c")
        sid = lax.axis_index("s")
        wid = sid * NC + cid
        pltpu.sync_copy(zeros_hbm.at[pl.ds(0, ZR)], zs_v)
        pltpu.sync_copy(zs_v, acc_sh.at[pl.ds(sid * ZR, ZR)])
        plsc.subcore_barrier()
        pltpu.sync_copy(row_hbm.at[wid], idx_v)
        pltpu.sync_copy(ew_hbm.at[wid], val_v)

        def body(j, carry):
            pltpu.sync_copy(val_v.at[j], acc_sh.at[idx_v.at[j]], add=True)
            return carry

        lax.fori_loop(0, cpt, body, 0)
        plsc.subcore_barrier()
        pltpu.sync_copy(acc_sh.at[pl.ds(sid * ZR, ZR)], zs_v)
        pltpu.sync_copy(zs_v, out_hbm.at[cid, pl.ds(sid * ZR, ZR)])

    return deg_kernel


def _make_agg(cpt, compute_wn):
    out_type = [jax.ShapeDtypeStruct((NC, N_PAD, HID), jnp.float32)]
    if compute_wn:
        out_type.append(jax.ShapeDtypeStruct((NW, cpt, CHUNK), jnp.float32))

    scratch = [
        pltpu.VMEM((cpt, CHUNK), jnp.int32),
        pltpu.VMEM((cpt, CHUNK), jnp.int32),
        pltpu.VMEM((cpt, CHUNK), jnp.float32),
        pltpu.VMEM((CHUNK, HID), jnp.float32),
        pltpu.VMEM((CHUNK, HID), jnp.float32),
        pltpu.VMEM((CHUNK, HID), jnp.float32),
        pltpu.VMEM((CHUNK, HID), jnp.float32),
        pltpu.VMEM((ZR, HID), jnp.float32),
        pltpu.VMEM_SHARED((N_PAD, HID), jnp.float32),
        pltpu.SemaphoreType.DMA,
        pltpu.SemaphoreType.DMA,
        pltpu.SemaphoreType.DMA,
        pltpu.SemaphoreType.DMA,
    ]
    if compute_wn:
        scratch.append(pltpu.VMEM((cpt, CHUNK), jnp.float32))
        scratch.append(pltpu.VMEM((N_PAD,), jnp.float32))

    @functools.partial(
        pl.kernel,
        mesh=_mesh,
        compiler_params=_sc_params,
        out_type=tuple(out_type) if compute_wn else out_type[0],
        scratch_types=scratch,
    )
    def agg_kernel(*refs):
        if compute_wn:
            (row_hbm, col_hbm, ew_hbm, dis_hbm, y_hbm, zeros_hbm,
             acc_out, wn_out,
             row_v, col_v, wn_v, rows_a, rows_b, sc_a, sc_b, st_v, acc_sh,
             sem_a, sem_b, sem_sa, sem_sb, ew_v, dis_v) = refs
        else:
            (row_hbm, col_hbm, wn_hbm, y_hbm, zeros_hbm,
             acc_out,
             row_v, col_v, wn_v, rows_a, rows_b, sc_a, sc_b, st_v, acc_sh,
             sem_a, sem_b, sem_sa, sem_sb) = refs
        cid = lax.axis_index("c")
        sid = lax.axis_index("s")
        wid = sid * NC + cid

        pltpu.sync_copy(zeros_hbm, st_v)
        pltpu.sync_copy(st_v, acc_sh.at[pl.ds(sid * ZR, ZR)])
        plsc.subcore_barrier()

        pltpu.sync_copy(row_hbm.at[wid], row_v)
        pltpu.sync_copy(col_hbm.at[wid], col_v)
        if compute_wn:
            pltpu.sync_copy(ew_hbm.at[wid], ew_v)
            pltpu.sync_copy(dis_hbm, dis_v)
        else:
            pltpu.sync_copy(wn_hbm.at[wid], wn_v)

        def compute_wn_chunk(j):
            @plsc.parallel_loop(0, CHUNK // L)
            def _(g):
                sl = pl.ds(g * L, L)
                dr = plsc.load_gather(dis_v, [row_v[j, sl]])
                dc = plsc.load_gather(dis_v, [col_v[j, sl]])
                wn_v[j, sl] = dr * ew_v[j, sl] * dc

        def scale(j, rows_v, sc_v):
            @plsc.parallel_loop(0, CHUNK, unroll=8)
            def _(i):
                wb = plsc.load_gather(wn_v, [_full16(j), _full16(i)])
                sc_v[i, :] = rows_v[i, :] * wb

        pltpu.async_copy(y_hbm.at[col_v.at[0]], rows_a, sem_a)

        def agg_body(p, carry):
            j0 = 2 * p
            j1 = j0 + 1
            pltpu.async_copy(y_hbm.at[col_v.at[j1]], rows_b, sem_b)
            if compute_wn:
                compute_wn_chunk(j0)
            pltpu.make_async_copy(y_hbm.at[col_v.at[j0]], rows_a, sem_a).wait()

            @pl.when(p > 0)
            def _():
                pltpu.make_async_copy(
                    sc_a, acc_sh.at[row_v.at[j0]], sem_sa).wait()

            scale(j0, rows_a, sc_a)
            pltpu.async_copy(sc_a, acc_sh.at[row_v.at[j0]], sem_sa, add=True)

            @pl.when(j1 + 1 < cpt)
            def _():
                pltpu.async_copy(y_hbm.at[col_v.at[j1 + 1]], rows_a, sem_a)

            if compute_wn:
                compute_wn_chunk(j1)
            pltpu.make_async_copy(y_hbm.at[col_v.at[j1]], rows_b, sem_b).wait()

            @pl.when(p > 0)
            def _():
                pltpu.make_async_copy(
                    sc_b, acc_sh.at[row_v.at[j1]], sem_sb).wait()

            scale(j1, rows_b, sc_b)
            pltpu.async_copy(sc_b, acc_sh.at[row_v.at[j1]], sem_sb, add=True)
            return carry

        lax.fori_loop(0, cpt // 2, agg_body, 0)
        pltpu.make_async_copy(sc_a, acc_sh.at[row_v.at[0]], sem_sa).wait()
        pltpu.make_async_copy(sc_b, acc_sh.at[row_v.at[0]], sem_sb).wait()
        if compute_wn:
            pltpu.sync_copy(wn_v, wn_out.at[wid])
        plsc.subcore_barrier()
        pltpu.sync_copy(acc_sh.at[pl.ds(sid * ZR, ZR)], st_v)
        pltpu.sync_copy(st_v, acc_out.at[cid, pl.ds(sid * ZR, ZR)])

    return agg_kernel


def _mm_kernel(x_ref, w_ref, t0_ref, y1_ref):
    o = jnp.dot(x_ref[...], w_ref[...], preferred_element_type=jnp.float32)
    t0_ref[...] = o[:, :HID]
    y1_ref[...] = o[:, HID:]


def _dis_kernel(deg_ref, o_ref):
    d = deg_ref[0, :] + deg_ref[1, :]
    o_ref[0, :] = jnp.where(
        d > 0.0, lax.rsqrt(jnp.maximum(d, 1e-30)), 0.0)


def _layer_out_kernel(t0_ref, acc_ref, b_ref, w_ref, u0_ref, y2_ref):
    h = t0_ref[...] - (acc_ref[0] + acc_ref[1]) + b_ref[...]
    h = jnp.maximum(h, 0.0)
    o = jnp.dot(h, w_ref[...], preferred_element_type=jnp.float32)
    u0_ref[...] = o[:, :C]
    y2_ref[...] = o[:, C:]


def _final_kernel(u0_ref, acc_ref, b_ref, o_ref):
    z = u0_ref[...] - (acc_ref[0] + acc_ref[1]) + b_ref[...]
    z = z - jnp.max(z, axis=1, keepdims=True)
    o_ref[...] = z - jnp.log(jnp.sum(jnp.exp(z), axis=1, keepdims=True))


_ROWS_BLK = 2000


def _tc_matmul(x, w):
    m, k = x.shape
    n = w.shape[1] // 2
    grid = m // _ROWS_BLK
    return pl.pallas_call(
        _mm_kernel,
        grid=(grid,),
        in_specs=[
            pl.BlockSpec((_ROWS_BLK, k), lambda i: (i, 0)),
            pl.BlockSpec((k, 2 * n), lambda i: (0, 0)),
        ],
        out_specs=[pl.BlockSpec((_ROWS_BLK, n), lambda i: (i, 0))] * 2,
        out_shape=[jax.ShapeDtypeStruct((m, n), jnp.float32)] * 2,
    )(x, w)


def _tc_dis(deg_p):
    return pl.pallas_call(
        _dis_kernel,
        out_shape=jax.ShapeDtypeStruct((1, N_PAD), jnp.float32),
    )(deg_p)


def _tc_layer_out(t0, acc_p, b, w):
    m = t0.shape[0]
    n = w.shape[1]
    grid = m // _ROWS_BLK
    return pl.pallas_call(
        _layer_out_kernel,
        grid=(grid,),
        in_specs=[
            pl.BlockSpec((_ROWS_BLK, HID), lambda i: (i, 0)),
            pl.BlockSpec((NC, _ROWS_BLK, HID), lambda i: (0, i, 0)),
            pl.BlockSpec((1, HID), lambda i: (0, 0)),
            pl.BlockSpec((HID, n), lambda i: (0, 0)),
        ],
        out_specs=[pl.BlockSpec((_ROWS_BLK, n // 2), lambda i: (i, 0))] * 2,
        out_shape=[jax.ShapeDtypeStruct((m, n // 2), jnp.float32)] * 2,
    )(t0, acc_p, b, w)


def _tc_final(u0, acc_p, b):
    m = u0.shape[0]
    grid = m // _ROWS_BLK
    return pl.pallas_call(
        _final_kernel,
        grid=(grid,),
        in_specs=[
            pl.BlockSpec((_ROWS_BLK, C), lambda i: (i, 0)),
            pl.BlockSpec((NC, _ROWS_BLK, C), lambda i: (0, i, 0)),
            pl.BlockSpec((1, C), lambda i: (0, 0)),
        ],
        out_specs=pl.BlockSpec((_ROWS_BLK, C), lambda i: (i, 0)),
        out_shape=jax.ShapeDtypeStruct((m, C), jnp.float32),
    )(u0, acc_p, b)


def kernel(x, edge_index, edge_weight, W1, b1, W2, b2):
    e = edge_index.shape[1]
    cpt = -(-e // (NW * CHUNK))
    cpt += cpt % 2
    e_pad = NW * cpt * CHUNK
    npad = e_pad - e

    row = edge_index[0]
    col = edge_index[1]
    k = jnp.arange(npad, dtype=jnp.int32)
    row_p = jnp.concatenate([row, N + (k % (N_PAD - N))]).reshape(NW, cpt, CHUNK)
    col_p = jnp.concatenate([col, (k * 97) % N]).reshape(NW, cpt, CHUNK)
    ew_p = jnp.concatenate(
        [edge_weight, jnp.zeros((npad,), jnp.float32)]).reshape(NW, cpt, CHUNK)
    zeros = jnp.zeros((ZR, HID), jnp.float32)

    deg_p = _make_deg(cpt)(row_p, ew_p, zeros[:, 0])
    dis = _tc_dis(deg_p)[0]

    w1c = jnp.concatenate([W1[0], W1[1]], axis=1)
    t0, y1 = _tc_matmul(x, w1c)

    acc1_p, wn = _make_agg(cpt, True)(row_p, col_p, ew_p, dis, y1, zeros)

    w2c = jnp.concatenate([W2[0], W2[1]], axis=1)
    u0, y2 = _tc_layer_out(t0, acc1_p, b1.reshape(1, HID), w2c)

    acc2_p = _make_agg(cpt, False)(row_p, col_p, wn, y2, zeros)

    return _tc_final(u0, acc2_p, b2.reshape(1, C))
